# baseline (speedup 1.0000x reference)
import dataclasses

import jax
import jax.numpy as jnp
from jax.experimental import pallas as pl
from jax.experimental.pallas import tpu as pltpu
from jax.experimental.pallas import tpu_sc as plsc

_EPS = 1e-12
_RB = 8
_LANES = 16


def _rsqrt_newton(x):
    i = jax.lax.bitcast_convert_type(x, jnp.int32)
    i = jnp.int32(0x5F3759DF) - (i >> 1)
    y = jax.lax.bitcast_convert_type(i, jnp.float32)
    for _ in range(3):
        y = y * (1.5 - 0.5 * x * y * y)
    return y


def kernel(word_embeddings, pos_table, ln_weight, ln_bias):
    batch, seq, hidden = word_embeddings.shape
    rows = batch * seq
    we2d = word_embeddings.reshape(rows, hidden)
    w2d = ln_weight.reshape(1, hidden)
    b2d = ln_bias.reshape(1, hidden)
    pos = pos_table[:seq]
    nchunks = hidden // _LANES
    inv_h = 1.0 / hidden
    pos_blocks = seq // _RB

    mesh = plsc.VectorSubcoreMesh(core_axis_name="c", subcore_axis_name="s")
    cp = pltpu.CompilerParams()
    if "needs_layout_passes" in pltpu.CompilerParams.__dataclass_fields__:
        cp = dataclasses.replace(cp, needs_layout_passes=False)

    @pl.kernel(
        out_type=jax.ShapeDtypeStruct((rows, hidden), jnp.float32),
        mesh=mesh,
        compiler_params=cp,
        scratch_types=[
            pltpu.VMEM((1, hidden), jnp.float32),
            pltpu.VMEM((1, hidden), jnp.float32),
            pltpu.VMEM((_LANES,), jnp.float32),
            pltpu.VMEM((_LANES,), jnp.float32),
        ],
    )
    def sc_kernel(we_hbm, pos_hbm, w_hbm, b_hbm, o_hbm, w_vmem, b_vmem,
                  acc1, acc2):
        pltpu.sync_copy(w_hbm, w_vmem)
        pltpu.sync_copy(b_hbm, b_vmem)

        def body(we_vmem, pos_vmem, out_vmem):
            @pl.loop(0, _RB)
            def _(r):
                acc1[...] = jnp.zeros((_LANES,), jnp.float32)
                acc2[...] = jnp.zeros((_LANES,), jnp.float32)

                @pl.loop(0, nchunks)
                def _(k):
                    sl = pl.ds(k * _LANES, _LANES)
                    v = we_vmem[r, sl] + pos_vmem[r, sl]
                    acc1[...] += v
                    acc2[...] += v * v

                s1 = jnp.sum(acc1[...])
                s2 = jnp.sum(acc2[...])
                mean = s1 * inv_h
                var = s2 * inv_h - mean * mean
                rs = _rsqrt_newton(var + _EPS)

                @pl.loop(0, nchunks)
                def _(k):
                    sl = pl.ds(k * _LANES, _LANES)
                    v = we_vmem[r, sl] + pos_vmem[r, sl]
                    out_vmem[r, sl] = (v - mean) * (rs * w_vmem[0, sl]) + b_vmem[0, sl]

        pltpu.emit_pipeline(
            body,
            grid=(rows // _RB,),
            in_specs=[
                pl.BlockSpec((_RB, hidden), lambda i: (i, 0)),
                pl.BlockSpec((_RB, hidden), lambda i: (i % pos_blocks, 0)),
            ],
            out_specs=[pl.BlockSpec((_RB, hidden), lambda i: (i, 0))],
            core_axis_name=("c", "s"),
            dimension_semantics=(pltpu.PARALLEL,),
        )(we_hbm, pos_hbm, o_hbm)

    out = sc_kernel(we2d, pos, w2d, b2d)
    return out.reshape(batch, seq, hidden)


# speedup vs baseline: 12.4202x; 12.4202x over previous
import jax
import jax.numpy as jnp
from jax.experimental import pallas as pl

_EPS = 1e-12
_SEQ_BLOCK = 512
_ROW_CHUNK = 128


def _ln_kernel(we_ref, pos_ref, w_ref, b_ref, out_ref):
    w = w_ref[...]
    b = b_ref[...]
    batch, s, hidden = we_ref.shape
    inv_h = 1.0 / hidden

    def body(i, _):
        r = i * _ROW_CHUNK
        for bi in range(batch):
            x = we_ref[bi, pl.ds(r, _ROW_CHUNK), :] + pos_ref[pl.ds(r, _ROW_CHUNK), :]
            s1 = jnp.sum(x, axis=-1, keepdims=True)
            s2 = jnp.sum(x * x, axis=-1, keepdims=True)
            mean = s1 * inv_h
            var = s2 * inv_h - mean * mean
            rs = jax.lax.rsqrt(var + _EPS)
            out_ref[bi, pl.ds(r, _ROW_CHUNK), :] = (
                (x - mean) * (rs * w) + b)
        return 0

    jax.lax.fori_loop(0, s // _ROW_CHUNK, body, 0)


def kernel(word_embeddings, pos_table, ln_weight, ln_bias):
    batch, seq, hidden = word_embeddings.shape
    s = _SEQ_BLOCK
    grid = (seq // s,)
    return pl.pallas_call(
        _ln_kernel,
        grid=grid,
        in_specs=[
            pl.BlockSpec((batch, s, hidden), lambda i: (0, i, 0)),
            pl.BlockSpec((s, hidden), lambda i: (i, 0)),
            pl.BlockSpec((hidden,), lambda i: (0,)),
            pl.BlockSpec((hidden,), lambda i: (0,)),
        ],
        out_specs=pl.BlockSpec((batch, s, hidden), lambda i: (0, i, 0)),
        out_shape=jax.ShapeDtypeStruct((batch, seq, hidden), jnp.float32),
    )(word_embeddings, pos_table[:seq], ln_weight, ln_bias)
